# baseline (speedup 1.0000x reference)
import functools

import jax
import jax.numpy as jnp
from jax import lax
from jax.experimental import pallas as pl
from jax.experimental.pallas import tpu as pltpu
from jax.experimental.pallas import tpu_sc as plsc

_EPS = float(jnp.finfo(jnp.float32).eps)

_N_ROWS = 100000
_DIM = 32


def _norm_body(tab_ref, nw_ref, out_ref):
    x = tab_ref[...]
    ms = jnp.mean(x * x, axis=-1, keepdims=True)
    out_ref[...] = x * lax.rsqrt(ms + _EPS) * nw_ref[...]


def _normalize_table(emb_weight, norm_weight):
    blk = 10000
    grid = _N_ROWS // blk
    return pl.pallas_call(
        _norm_body,
        grid=(grid,),
        in_specs=[
            pl.BlockSpec((blk, _DIM), lambda i: (i, 0)),
            pl.BlockSpec((1, _DIM), lambda i: (0, 0)),
        ],
        out_specs=pl.BlockSpec((blk, _DIM), lambda i: (i, 0)),
        out_shape=jax.ShapeDtypeStruct((_N_ROWS, _DIM), jnp.float32),
    )(emb_weight, norm_weight.reshape(1, _DIM))


def _make_gather(b, h):
    info = plsc.get_sparse_core_info()
    nc, ns = info.num_cores, info.num_subcores
    nw = nc * ns
    nbt = b // 128
    irows = h * nbt
    per_w = irows // nw
    k = 4
    ch = k * 128
    steps = per_w // k
    assert nbt % k == 0 and per_w % k == 0 and steps >= 4

    mesh = plsc.VectorSubcoreMesh(core_axis_name="c", subcore_axis_name="s")

    @functools.partial(
        pl.kernel,
        mesh=mesh,
        compiler_params=pltpu.CompilerParams(
            use_tc_tiling_on_sc=False,
            needs_layout_passes=False,
            disable_bounds_checks=True,
        ),
        out_type=jax.ShapeDtypeStruct((h * 4 * nbt * 8 * 128,), jnp.float32),
        scratch_types=[
            pltpu.VMEM((k, 128), jnp.int32),
            pltpu.VMEM((k, 128), jnp.int32),
            pltpu.VMEM((ch, _DIM), jnp.float32),
            pltpu.VMEM((ch, _DIM), jnp.float32),
            pltpu.VMEM((4 * k * 8 * 128,), jnp.float32),
            pltpu.VMEM((4 * k * 8 * 128,), jnp.float32),
            pltpu.SemaphoreType.DMA,
            pltpu.SemaphoreType.DMA,
            pltpu.SemaphoreType.DMA,
            pltpu.SemaphoreType.DMA,
            pltpu.SemaphoreType.DMA,
            pltpu.SemaphoreType.DMA,
        ],
    )
    def gather(tab_hbm, idx_hbm, out_hbm, i0, i1, g0, g1, t0, t1,
               is0, is1, gs0, gs1, os0, os1):
        wid = lax.axis_index("s") * nc + lax.axis_index("c")
        r0 = wid * per_w
        ibuf, gbuf, tbuf = (i0, i1), (g0, g1), (t0, t1)
        isem, gsem, osem = (is0, is1), (gs0, gs1), (os0, os1)

        def icp(i, s):
            return pltpu.async_copy(
                idx_hbm.at[pl.ds(r0 + i * k, k)], ibuf[s], isem[s]
            )

        def fire_gathers(i, s):
            for j in range(k):
                pltpu.async_copy(
                    tab_hbm.at[ibuf[s].at[j]],
                    gbuf[s].at[pl.ds(j * 128, 128)],
                    gsem[s],
                )

        def drain_gathers(s):
            for j in range(k):
                pltpu.make_async_copy(
                    tab_hbm.at[ibuf[s].at[j]],
                    gbuf[s].at[pl.ds(j * 128, 128)],
                    gsem[s],
                ).wait()

        def transpose(s):
            iota = lax.iota(jnp.int32, 16)

            @plsc.parallel_loop(0, _DIM)
            def _(d):
                fv = (d + iota) & 31
                pv = ((fv >> 3) << 12) + ((fv & 7) << 7) + iota

                @plsc.parallel_loop(0, ch // 16, unroll=16)
                def _(g):
                    rowv = g * 16 + iota
                    bg = ((g >> 3) << 10) + ((g & 7) << 4)
                    v = plsc.load_gather(gbuf[s], [rowv, fv])
                    plsc.store_scatter(tbuf[s], [pv + bg], v)

        def ocp(i, s):
            r = r0 + i * k
            hh = r // 128
            bt0 = lax.rem(r, 128)
            obase = hh * (4 * nbt * 1024) + bt0 * 1024
            for ft in range(4):
                pltpu.async_copy(
                    tbuf[s].at[pl.ds(ft * (k * 1024), k * 1024)],
                    out_hbm.at[pl.ds(obase + ft * (nbt * 1024), k * 1024)],
                    osem[s],
                )

        def ocp_wait(s):
            for ft in range(4):
                pltpu.make_async_copy(
                    tbuf[s].at[pl.ds(ft * (k * 1024), k * 1024)],
                    out_hbm.at[pl.ds(0, k * 1024)],
                    osem[s],
                ).wait()

        def icp_wait(s):
            pltpu.make_async_copy(
                idx_hbm.at[pl.ds(0, k)], ibuf[s], isem[s]
            ).wait()

        icp(0, 0)
        icp(1, 1)
        icp_wait(0)
        fire_gathers(0, 0)
        icp_wait(1)
        fire_gathers(1, 1)
        drain_gathers(0)
        transpose(0)
        icp(2, 0)
        ocp(0, 0)

        def body(p, carry):
            icp_wait(0)
            fire_gathers(2 * p + 2, 0)
            drain_gathers(1)
            icp(2 * p + 3, 1)

            @pl.when(p > 0)
            def _():
                ocp_wait(1)

            transpose(1)
            ocp(2 * p + 1, 1)
            icp_wait(1)
            fire_gathers(2 * p + 3, 1)
            drain_gathers(0)

            @pl.when(p < steps // 2 - 2)
            def _():
                icp(2 * p + 4, 0)

            ocp_wait(0)
            transpose(0)
            ocp(2 * p + 2, 0)
            return carry

        lax.fori_loop(0, steps // 2 - 1, body, 0)

        drain_gathers(1)
        ocp_wait(1)
        transpose(1)
        ocp(steps - 1, 1)
        ocp_wait(0)
        ocp_wait(1)

    return gather


def kernel(component_labels, emb_weight, norm_weight):
    b, h = component_labels.shape
    tab = _normalize_table(emb_weight, norm_weight)
    idx_t = component_labels.astype(jnp.int32).T.reshape(h * (b // 128), 128)
    out_flat = _make_gather(b, h)(tab, idx_t)
    out5 = out_flat.reshape(h, 4, b // 128, 8, 128)
    return out5.transpose(2, 4, 0, 1, 3).reshape(b, h, _DIM)


# speedup vs baseline: 1.1470x; 1.1470x over previous
import functools

import jax
import jax.numpy as jnp
from jax import lax
from jax.experimental import pallas as pl
from jax.experimental.pallas import tpu as pltpu
from jax.experimental.pallas import tpu_sc as plsc

_EPS = float(jnp.finfo(jnp.float32).eps)

_N_ROWS = 100000
_DIM = 32


def _norm_body(tabt_ref, nw_ref, out_ref):
    x = tabt_ref[...]
    ms = jnp.mean(x * x, axis=0, keepdims=True)
    xn = x * lax.rsqrt(ms + _EPS)
    out_ref[...] = xn.T * nw_ref[...]


def _normalize_table(emb_weight_t, norm_weight):
    return pl.pallas_call(
        _norm_body,
        compiler_params=pltpu.CompilerParams(
            vmem_limit_bytes=100 * 1024 * 1024
        ),
        in_specs=[
            pl.BlockSpec((_DIM, _N_ROWS), lambda: (0, 0)),
            pl.BlockSpec((1, _DIM), lambda: (0, 0)),
        ],
        out_specs=pl.BlockSpec((_N_ROWS, _DIM), lambda: (0, 0)),
        out_shape=jax.ShapeDtypeStruct((_N_ROWS, _DIM), jnp.float32),
    )(emb_weight_t, norm_weight.reshape(1, _DIM))


def _make_gather(b, h):
    info = plsc.get_sparse_core_info()
    nc, ns = info.num_cores, info.num_subcores
    nw = nc * ns
    nbt = b // 128
    irows = h * nbt
    per_w = irows // nw
    k = 4
    ch = k * 128
    steps = per_w // k
    assert nbt % k == 0 and per_w % k == 0 and steps >= 4

    mesh = plsc.VectorSubcoreMesh(core_axis_name="c", subcore_axis_name="s")

    @functools.partial(
        pl.kernel,
        mesh=mesh,
        compiler_params=pltpu.CompilerParams(
            use_tc_tiling_on_sc=False,
            needs_layout_passes=False,
            disable_bounds_checks=True,
        ),
        out_type=jax.ShapeDtypeStruct((h * 4 * nbt * 8 * 128,), jnp.float32),
        scratch_types=[
            pltpu.VMEM((k, 128), jnp.int32),
            pltpu.VMEM((k, 128), jnp.int32),
            pltpu.VMEM((ch, _DIM), jnp.float32),
            pltpu.VMEM((ch, _DIM), jnp.float32),
            pltpu.VMEM((4 * k * 8 * 128,), jnp.float32),
            pltpu.VMEM((4 * k * 8 * 128,), jnp.float32),
            pltpu.SemaphoreType.DMA,
            pltpu.SemaphoreType.DMA,
            pltpu.SemaphoreType.DMA,
            pltpu.SemaphoreType.DMA,
            pltpu.SemaphoreType.DMA,
            pltpu.SemaphoreType.DMA,
        ],
    )
    def gather(tab_hbm, idx_hbm, out_hbm, i0, i1, g0, g1, t0, t1,
               is0, is1, gs0, gs1, os0, os1):
        wid = lax.axis_index("s") * nc + lax.axis_index("c")
        r0 = wid * per_w
        ibuf, gbuf, tbuf = (i0, i1), (g0, g1), (t0, t1)
        isem, gsem, osem = (is0, is1), (gs0, gs1), (os0, os1)

        def icp(i, s):
            return pltpu.async_copy(
                idx_hbm.at[pl.ds(r0 + i * k, k)], ibuf[s], isem[s]
            )

        def fire_gathers(i, s):
            for j in range(k):
                pltpu.async_copy(
                    tab_hbm.at[ibuf[s].at[j]],
                    gbuf[s].at[pl.ds(j * 128, 128)],
                    gsem[s],
                )

        def drain_gathers(s):
            for j in range(k):
                pltpu.make_async_copy(
                    tab_hbm.at[ibuf[s].at[j]],
                    gbuf[s].at[pl.ds(j * 128, 128)],
                    gsem[s],
                ).wait()

        def transpose(s):
            iota = lax.iota(jnp.int32, 16)

            @plsc.parallel_loop(0, _DIM)
            def _(d):
                fv = (d + iota) & 31
                pv = ((fv >> 3) << 12) + ((fv & 7) << 7) + iota

                @plsc.parallel_loop(0, ch // 16, unroll=8)
                def _(g):
                    rowv = g * 16 + iota
                    bg = ((g >> 3) << 10) + ((g & 7) << 4)
                    v = plsc.load_gather(gbuf[s], [rowv, fv])
                    plsc.store_scatter(tbuf[s], [pv + bg], v)

        def ocp(i, s):
            r = r0 + i * k
            hh = r // 128
            bt0 = lax.rem(r, 128)
            obase = hh * (4 * nbt * 1024) + bt0 * 1024
            for ft in range(4):
                pltpu.async_copy(
                    tbuf[s].at[pl.ds(ft * (k * 1024), k * 1024)],
                    out_hbm.at[pl.ds(obase + ft * (nbt * 1024), k * 1024)],
                    osem[s],
                )

        def ocp_wait(s):
            for ft in range(4):
                pltpu.make_async_copy(
                    tbuf[s].at[pl.ds(ft * (k * 1024), k * 1024)],
                    out_hbm.at[pl.ds(0, k * 1024)],
                    osem[s],
                ).wait()

        def icp_wait(s):
            pltpu.make_async_copy(
                idx_hbm.at[pl.ds(0, k)], ibuf[s], isem[s]
            ).wait()

        icp(0, 0)
        icp(1, 1)
        icp_wait(0)
        fire_gathers(0, 0)
        icp_wait(1)
        fire_gathers(1, 1)
        drain_gathers(0)
        transpose(0)
        icp(2, 0)
        ocp(0, 0)

        def body(p, carry):
            icp_wait(0)
            fire_gathers(2 * p + 2, 0)
            drain_gathers(1)
            icp(2 * p + 3, 1)

            @pl.when(p > 0)
            def _():
                ocp_wait(1)

            transpose(1)
            ocp(2 * p + 1, 1)
            icp_wait(1)
            fire_gathers(2 * p + 3, 1)
            drain_gathers(0)

            @pl.when(p < steps // 2 - 2)
            def _():
                icp(2 * p + 4, 0)

            ocp_wait(0)
            transpose(0)
            ocp(2 * p + 2, 0)
            return carry

        lax.fori_loop(0, steps // 2 - 1, body, 0)

        drain_gathers(1)
        ocp_wait(1)
        transpose(1)
        ocp(steps - 1, 1)
        ocp_wait(0)
        ocp_wait(1)

    return gather


def kernel(component_labels, emb_weight, norm_weight):
    b, h = component_labels.shape
    tab = _normalize_table(emb_weight.T, norm_weight)
    idx_t = component_labels.astype(jnp.int32).T.reshape(h * (b // 128), 128)
    out_flat = _make_gather(b, h)(tab, idx_t)
    out5 = out_flat.reshape(h, 4, b // 128, 8, 128)
    return out5.transpose(2, 4, 0, 1, 3).reshape(b, h, _DIM)


# speedup vs baseline: 1.1494x; 1.0020x over previous
import functools

import jax
import jax.numpy as jnp
from jax import lax
from jax.experimental import pallas as pl
from jax.experimental.pallas import tpu as pltpu
from jax.experimental.pallas import tpu_sc as plsc

_EPS = float(jnp.finfo(jnp.float32).eps)

_N_ROWS = 100000
_DIM = 32


def _norm_body(tabt_ref, nw_ref, out_ref):
    x = tabt_ref[...]
    ms = jnp.mean(x * x, axis=0, keepdims=True)
    xn = x * lax.rsqrt(ms + _EPS)
    out_ref[...] = xn.T * nw_ref[...]


def _normalize_table(emb_weight_t, norm_weight):
    return pl.pallas_call(
        _norm_body,
        compiler_params=pltpu.CompilerParams(
            vmem_limit_bytes=100 * 1024 * 1024
        ),
        in_specs=[
            pl.BlockSpec((_DIM, _N_ROWS), lambda: (0, 0)),
            pl.BlockSpec((1, _DIM), lambda: (0, 0)),
        ],
        out_specs=pl.BlockSpec((_N_ROWS, _DIM), lambda: (0, 0)),
        out_shape=jax.ShapeDtypeStruct((_N_ROWS, _DIM), jnp.float32),
    )(emb_weight_t, norm_weight.reshape(1, _DIM))


def _make_gather(b, h):
    info = plsc.get_sparse_core_info()
    nc, ns = info.num_cores, info.num_subcores
    nw = nc * ns
    nbt = b // 128
    irows = h * nbt
    per_w = irows // nw
    k = 4
    ch = k * 128
    steps = per_w // k
    assert nbt % k == 0 and per_w % k == 0 and steps >= 4

    mesh = plsc.VectorSubcoreMesh(core_axis_name="c", subcore_axis_name="s")

    @functools.partial(
        pl.kernel,
        mesh=mesh,
        compiler_params=pltpu.CompilerParams(
            use_tc_tiling_on_sc=False,
            needs_layout_passes=False,
            disable_bounds_checks=True,
        ),
        out_type=jax.ShapeDtypeStruct((h * 4 * nbt * 8 * 128,), jnp.float32),
        scratch_types=[
            pltpu.VMEM((k, 128), jnp.int32),
            pltpu.VMEM((k, 128), jnp.int32),
            pltpu.VMEM((ch, _DIM), jnp.float32),
            pltpu.VMEM((ch, _DIM), jnp.float32),
            pltpu.VMEM((4 * k * 8 * 128,), jnp.float32),
            pltpu.VMEM((4 * k * 8 * 128,), jnp.float32),
            pltpu.SemaphoreType.DMA,
            pltpu.SemaphoreType.DMA,
            pltpu.SemaphoreType.DMA,
            pltpu.SemaphoreType.DMA,
            pltpu.SemaphoreType.DMA,
            pltpu.SemaphoreType.DMA,
        ],
    )
    def gather(tab_hbm, idx_hbm, out_hbm, i0, i1, g0, g1, t0, t1,
               is0, is1, gs0, gs1, os0, os1):
        wid = lax.axis_index("s") * nc + lax.axis_index("c")
        r0 = wid * per_w
        ibuf, gbuf, tbuf = (i0, i1), (g0, g1), (t0, t1)
        isem, gsem, osem = (is0, is1), (gs0, gs1), (os0, os1)

        def icp(i, s):
            return pltpu.async_copy(
                idx_hbm.at[pl.ds(r0 + i * k, k)], ibuf[s], isem[s]
            )

        def fire_gathers(i, s):
            for j in range(k):
                pltpu.async_copy(
                    tab_hbm.at[ibuf[s].at[j]],
                    gbuf[s].at[pl.ds(j * 128, 128)],
                    gsem[s],
                )

        def drain_gathers(s):
            pltpu.make_async_copy(
                tab_hbm.at[pl.ds(0, ch)], gbuf[s], gsem[s]
            ).wait()

        def transpose(s):
            iota = lax.iota(jnp.int32, 16)

            @plsc.parallel_loop(0, _DIM)
            def _(d):
                fv = (d + iota) & 31
                pv = ((fv >> 3) << 12) + ((fv & 7) << 7) + iota

                @plsc.parallel_loop(0, ch // 16, unroll=8)
                def _(g):
                    rowv = g * 16 + iota
                    bg = ((g >> 3) << 10) + ((g & 7) << 4)
                    v = plsc.load_gather(gbuf[s], [rowv, fv])
                    plsc.store_scatter(tbuf[s], [pv + bg], v)

        def ocp(i, s):
            r = r0 + i * k
            hh = r // 128
            bt0 = lax.rem(r, 128)
            obase = hh * (4 * nbt * 1024) + bt0 * 1024
            for ft in range(4):
                pltpu.async_copy(
                    tbuf[s].at[pl.ds(ft * (k * 1024), k * 1024)],
                    out_hbm.at[pl.ds(obase + ft * (nbt * 1024), k * 1024)],
                    osem[s],
                )

        def ocp_wait(s):
            pltpu.make_async_copy(
                tbuf[s], out_hbm.at[pl.ds(0, 4 * k * 1024)], osem[s]
            ).wait()

        def icp_wait(s):
            pltpu.make_async_copy(
                idx_hbm.at[pl.ds(0, k)], ibuf[s], isem[s]
            ).wait()

        icp(0, 0)
        icp(1, 1)
        icp_wait(0)
        fire_gathers(0, 0)
        icp_wait(1)
        fire_gathers(1, 1)
        drain_gathers(0)
        transpose(0)
        icp(2, 0)
        ocp(0, 0)

        def body(p, carry):
            icp_wait(0)
            fire_gathers(2 * p + 2, 0)
            drain_gathers(1)
            icp(2 * p + 3, 1)

            @pl.when(p > 0)
            def _():
                ocp_wait(1)

            transpose(1)
            ocp(2 * p + 1, 1)
            icp_wait(1)
            fire_gathers(2 * p + 3, 1)
            drain_gathers(0)

            @pl.when(p < steps // 2 - 2)
            def _():
                icp(2 * p + 4, 0)

            ocp_wait(0)
            transpose(0)
            ocp(2 * p + 2, 0)
            return carry

        lax.fori_loop(0, steps // 2 - 1, body, 0)

        drain_gathers(1)
        ocp_wait(1)
        transpose(1)
        ocp(steps - 1, 1)
        ocp_wait(0)
        ocp_wait(1)

    return gather


def kernel(component_labels, emb_weight, norm_weight):
    b, h = component_labels.shape
    tab = _normalize_table(emb_weight.T, norm_weight)
    idx_t = component_labels.astype(jnp.int32).T.reshape(h * (b // 128), 128)
    out_flat = _make_gather(b, h)(tab, idx_t)
    out5 = out_flat.reshape(h, 4, b // 128, 8, 128)
    return out5.transpose(2, 4, 0, 1, 3).reshape(b, h, _DIM)


# speedup vs baseline: 1.2070x; 1.0501x over previous
import functools

import jax
import jax.numpy as jnp
from jax import lax
from jax.experimental import pallas as pl
from jax.experimental.pallas import tpu as pltpu
from jax.experimental.pallas import tpu_sc as plsc

_EPS = float(jnp.finfo(jnp.float32).eps)

_N_ROWS = 100000
_DIM = 32


def _norm_body(tabt_ref, nw_ref, out_ref):
    x = tabt_ref[...]
    ms = jnp.mean(x * x, axis=0, keepdims=True)
    xn = x * lax.rsqrt(ms + _EPS)
    out_ref[...] = xn.T * nw_ref[...]


def _normalize_table(emb_weight_t, norm_weight):
    return pl.pallas_call(
        _norm_body,
        compiler_params=pltpu.CompilerParams(
            vmem_limit_bytes=100 * 1024 * 1024
        ),
        in_specs=[
            pl.BlockSpec((_DIM, _N_ROWS), lambda: (0, 0)),
            pl.BlockSpec((1, _DIM), lambda: (0, 0)),
        ],
        out_specs=pl.BlockSpec((_N_ROWS, _DIM), lambda: (0, 0)),
        out_shape=jax.ShapeDtypeStruct((_N_ROWS, _DIM), jnp.float32),
    )(emb_weight_t, norm_weight.reshape(1, _DIM))


def _make_gather(b, h):
    info = plsc.get_sparse_core_info()
    nc, ns = info.num_cores, info.num_subcores
    nw = nc * ns
    nbt = b // 128
    irows = h * nbt
    per_w = irows // nw
    k = 4
    ch = k * 128
    steps = per_w // k
    assert nbt % k == 0 and per_w % k == 0 and steps >= 4

    mesh = plsc.VectorSubcoreMesh(core_axis_name="c", subcore_axis_name="s")

    @functools.partial(
        pl.kernel,
        mesh=mesh,
        compiler_params=pltpu.CompilerParams(
            use_tc_tiling_on_sc=False,
            needs_layout_passes=False,
            disable_bounds_checks=True,
        ),
        out_type=jax.ShapeDtypeStruct((h * 4 * nbt * 8 * 128,), jnp.float32),
        scratch_types=[
            pltpu.VMEM((k, 128), jnp.int32),
            pltpu.VMEM((k, 128), jnp.int32),
            pltpu.VMEM((ch, _DIM), jnp.float32),
            pltpu.VMEM((ch, _DIM), jnp.float32),
            pltpu.VMEM((4 * k * 8 * 128,), jnp.float32),
            pltpu.VMEM((4 * k * 8 * 128,), jnp.float32),
            pltpu.SemaphoreType.DMA,
            pltpu.SemaphoreType.DMA,
            pltpu.SemaphoreType.DMA,
            pltpu.SemaphoreType.DMA,
            pltpu.SemaphoreType.DMA,
            pltpu.SemaphoreType.DMA,
        ],
    )
    def gather(tab_hbm, idx_hbm, out_hbm, i0, i1, g0, g1, t0, t1,
               is0, is1, gs0, gs1, os0, os1):
        wid = lax.axis_index("s") * nc + lax.axis_index("c")
        r0 = wid * per_w
        ibuf, gbuf, tbuf = (i0, i1), (g0, g1), (t0, t1)
        isem, gsem, osem = (is0, is1), (gs0, gs1), (os0, os1)

        def icp(i, s):
            return pltpu.async_copy(
                idx_hbm.at[pl.ds(r0 + i * k, k)], ibuf[s], isem[s]
            )

        def fire_gathers(i, s):
            for j in range(k):
                pltpu.async_copy(
                    tab_hbm.at[ibuf[s].at[j]],
                    gbuf[s].at[pl.ds(j * 128, 128)],
                    gsem[s],
                )

        def drain_gathers(s):
            pltpu.make_async_copy(
                tab_hbm.at[pl.ds(0, ch)], gbuf[s], gsem[s]
            ).wait()

        def transpose(s):
            iota = lax.iota(jnp.int32, 16)

            zero = jnp.zeros((16,), jnp.int32)

            @plsc.parallel_loop(0, _DIM)
            def _(d):
                fv = (d + iota) & 31
                qv = (iota << 5) + fv
                pv = ((fv >> 3) << 12) + ((fv & 7) << 7) + iota

                @plsc.parallel_loop(0, ch // 16, unroll=8)
                def _(g):
                    bg = ((g >> 3) << 10) + ((g & 7) << 4)
                    v = plsc.load_gather(gbuf[s], [zero, qv + (g << 9)])
                    plsc.store_scatter(tbuf[s], [pv + bg], v)

        def ocp(i, s):
            r = r0 + i * k
            hh = r // 128
            bt0 = lax.rem(r, 128)
            obase = hh * (4 * nbt * 1024) + bt0 * 1024
            for ft in range(4):
                pltpu.async_copy(
                    tbuf[s].at[pl.ds(ft * (k * 1024), k * 1024)],
                    out_hbm.at[pl.ds(obase + ft * (nbt * 1024), k * 1024)],
                    osem[s],
                )

        def ocp_wait(s):
            pltpu.make_async_copy(
                tbuf[s], out_hbm.at[pl.ds(0, 4 * k * 1024)], osem[s]
            ).wait()

        def icp_wait(s):
            pltpu.make_async_copy(
                idx_hbm.at[pl.ds(0, k)], ibuf[s], isem[s]
            ).wait()

        icp(0, 0)
        icp(1, 1)
        icp_wait(0)
        fire_gathers(0, 0)
        icp_wait(1)
        fire_gathers(1, 1)
        drain_gathers(0)
        transpose(0)
        icp(2, 0)
        ocp(0, 0)

        def body(p, carry):
            icp_wait(0)
            fire_gathers(2 * p + 2, 0)
            drain_gathers(1)
            icp(2 * p + 3, 1)

            @pl.when(p > 0)
            def _():
                ocp_wait(1)

            transpose(1)
            ocp(2 * p + 1, 1)
            icp_wait(1)
            fire_gathers(2 * p + 3, 1)
            drain_gathers(0)

            @pl.when(p < steps // 2 - 2)
            def _():
                icp(2 * p + 4, 0)

            ocp_wait(0)
            transpose(0)
            ocp(2 * p + 2, 0)
            return carry

        lax.fori_loop(0, steps // 2 - 1, body, 0)

        drain_gathers(1)
        ocp_wait(1)
        transpose(1)
        ocp(steps - 1, 1)
        ocp_wait(0)
        ocp_wait(1)

    return gather


def kernel(component_labels, emb_weight, norm_weight):
    b, h = component_labels.shape
    tab = _normalize_table(emb_weight.T, norm_weight)
    idx_t = component_labels.astype(jnp.int32).T.reshape(h * (b // 128), 128)
    out_flat = _make_gather(b, h)(tab, idx_t)
    out5 = out_flat.reshape(h, 4, b // 128, 8, 128)
    return out5.transpose(2, 4, 0, 1, 3).reshape(b, h, _DIM)
